# baseline (speedup 1.0000x reference)
import functools

import jax
import jax.numpy as jnp
from jax import lax
from jax.experimental import pallas as pl
from jax.experimental.pallas import tpu as pltpu
from jax.experimental.pallas import tpu_sc as plsc

N = 10000
E = 320000
D_IN = 128
D_HID = 64
D_LAT = 32
N_GRAPHS = 16

NC = 2
NS = 16
NW = NC * NS
CH = 128
NCH = 80
K4 = 4
PADE = NW * NCH * CH
NACC = 10240
ZROWS = 64
BN = 1000
GRID = N // BN

@functools.lru_cache(maxsize=None)
def _get_mesh():
    return plsc.VectorSubcoreMesh(core_axis_name="c", subcore_axis_name="s",
                                  num_cores=NC, num_subcores=NS)


@functools.lru_cache(maxsize=None)
def _make_sc_scatter(D):
    rows_per_tile_out = NACC // NS
    zcopies = NACC // NS // ZROWS

    @functools.partial(
        pl.kernel,
        out_type=jax.ShapeDtypeStruct((NC, NACC, D), jnp.float32),
        mesh=_get_mesh(),
        compiler_params=pltpu.CompilerParams(use_tc_tiling_on_sc=False),
        scratch_types=[
            pltpu.VMEM((NCH, CH), jnp.int32),
            pltpu.VMEM((NCH, CH), jnp.int32),
            pltpu.VMEM((2 * K4 * CH, D), jnp.float32),
            pltpu.VMEM((ZROWS, D), jnp.float32),
            pltpu.VMEM_SHARED((NACC, D), jnp.float32),
            pltpu.SemaphoreType.DMA((2 * K4,)),
        ],
    )
    def k(src3, dst3, g_hbm, out, src_v, dst_v, rows_v, zero_v, acc_sh, gsem):
        c = lax.axis_index("c")
        s = lax.axis_index("s")
        wid = s * NC + c

        pltpu.sync_copy(src3.at[wid], src_v)
        pltpu.sync_copy(dst3.at[wid], dst_v)

        def zb(i, _):
            zero_v[i, :] = jnp.zeros((D,), jnp.float32)
            return 0
        lax.fori_loop(0, ZROWS, zb, 0)

        def zc(t, _):
            pltpu.sync_copy(zero_v, acc_sh.at[pl.ds(s * (NACC // NS) + t * ZROWS, ZROWS)])
            return 0
        lax.fori_loop(0, zcopies, zc, 0)
        plsc.subcore_barrier()

        def buf(b):
            return rows_v.at[pl.ds(b * CH, CH)]

        def fire(j, b):
            pltpu.async_copy(g_hbm.at[src_v.at[j]], buf(b), gsem.at[b])

        def wait_g(b):
            pltpu.make_async_copy(g_hbm.at[src_v.at[0]], buf(b), gsem.at[b]).wait()

        def scat(j, b):
            pltpu.sync_copy(buf(b), acc_sh.at[dst_v.at[j]], add=True)

        for b in range(K4):
            fire(b, b)

        @pl.loop(0, NCH, step=2 * K4)
        def _pair(j0):
            for b in range(K4):
                fire(j0 + K4 + b, K4 + b)
            for b in range(K4):
                wait_g(b)
                scat(j0 + b, b)

            @pl.when(j0 + 2 * K4 < NCH)
            def _():
                for b in range(K4):
                    fire(j0 + 2 * K4 + b, b)

            for b in range(K4):
                wait_g(K4 + b)
                scat(j0 + K4 + b, K4 + b)

        plsc.subcore_barrier()

        r0 = s * rows_per_tile_out
        pltpu.sync_copy(acc_sh.at[pl.ds(r0, rows_per_tile_out)],
                        out.at[c, pl.ds(r0, rows_per_tile_out)])

    return k


_DDEG = 8


@functools.lru_cache(maxsize=None)
def _make_sc_degree():
    @functools.partial(
        pl.kernel,
        out_type=jax.ShapeDtypeStruct((NC, NACC, _DDEG), jnp.float32),
        mesh=_get_mesh(),
        compiler_params=pltpu.CompilerParams(use_tc_tiling_on_sc=False),
        scratch_types=[
            pltpu.VMEM((NCH, CH), jnp.int32),
            pltpu.VMEM((CH, _DDEG), jnp.float32),
            pltpu.VMEM((ZROWS, _DDEG), jnp.float32),
            pltpu.VMEM_SHARED((NACC, _DDEG), jnp.float32),
        ],
    )
    def k(dst3, ones_hbm, out, dst_v, ones_v, zero_v, acc_sh):
        c = lax.axis_index("c")
        s = lax.axis_index("s")
        wid = s * NC + c
        rows_per_tile_out = NACC // NS
        zcopies = NACC // NS // ZROWS

        pltpu.sync_copy(dst3.at[wid], dst_v)
        pltpu.sync_copy(ones_hbm, ones_v)

        def zb(i, _):
            zero_v[i, :] = jnp.zeros((_DDEG,), jnp.float32)
            return 0
        lax.fori_loop(0, ZROWS, zb, 0)

        def zc(t, _):
            pltpu.sync_copy(zero_v, acc_sh.at[pl.ds(s * (NACC // NS) + t * ZROWS, ZROWS)])
            return 0
        lax.fori_loop(0, zcopies, zc, 0)
        plsc.subcore_barrier()

        def body(j, _):
            pltpu.sync_copy(ones_v, acc_sh.at[dst_v.at[j]], add=True)
            return 0
        lax.fori_loop(0, NCH, body, 0)
        plsc.subcore_barrier()

        r0 = s * rows_per_tile_out
        pltpu.sync_copy(acc_sh.at[pl.ds(r0, rows_per_tile_out)],
                        out.at[c, pl.ds(r0, rows_per_tile_out)])

    return k


def _row_spec(d):
    return pl.BlockSpec((BN, d), lambda i: (i, 0))


def _acc_spec(d):
    return pl.BlockSpec((NC, BN, d), lambda i: (0, i, 0))


def _full_spec(shape):
    return pl.BlockSpec(shape, lambda i: (0,) * len(shape))


def _tc1_body(x_ref, w_ref, deg_ref, g1_ref, dis_ref):
    deg = deg_ref[0, :, 0:1] + deg_ref[1, :, 0:1] + 1.0
    dis = lax.rsqrt(deg)
    dis_ref[...] = jnp.broadcast_to(dis, (BN, _DDEG))
    h = jnp.dot(x_ref[...], w_ref[...], preferred_element_type=jnp.float32)
    g1_ref[...] = h * dis


def _tc1(x, W_e1, deg_part):
    return pl.pallas_call(
        _tc1_body,
        grid=(GRID,),
        in_specs=[_row_spec(D_IN), _full_spec((D_IN, D_HID)), _acc_spec(_DDEG)],
        out_specs=[_row_spec(D_HID), _row_spec(_DDEG)],
        out_shape=[jax.ShapeDtypeStruct((N, D_HID), jnp.float32),
                   jax.ShapeDtypeStruct((N, _DDEG), jnp.float32)],
    )(x, W_e1, deg_part)


def _tc2_body(acc_ref, g_ref, dis_ref, b_ref, w_ref, out_ref):
    dis = dis_ref[:, 0:1]
    conv = dis * (acc_ref[0] + acc_ref[1] + g_ref[...]) + b_ref[0]
    h = jnp.maximum(conv, 0.0)
    out_ref[...] = jnp.dot(h, w_ref[...], preferred_element_type=jnp.float32) * dis


def _tc2(acc1, g1, dis8, b_e1, W_e2):
    return pl.pallas_call(
        _tc2_body,
        grid=(GRID,),
        in_specs=[_acc_spec(D_HID), _row_spec(D_HID), _row_spec(_DDEG),
                  _full_spec((1, D_HID)), _full_spec((D_HID, D_LAT))],
        out_specs=[_row_spec(D_LAT)],
        out_shape=[jax.ShapeDtypeStruct((N, D_LAT), jnp.float32)],
    )(acc1, g1, dis8, b_e1.reshape(1, D_HID), W_e2)[0]


def _tc3_body(acc_ref, g_ref, dis_ref, b_ref, batch_ref,
              z_ref, g3_ref, agg_ref, sums_ref, cnt_ref):
    i = pl.program_id(0)
    dis = dis_ref[:, 0:1]
    z = dis * (acc_ref[0] + acc_ref[1] + g_ref[...]) + b_ref[0]
    z_ref[...] = z
    g3_ref[...] = z * dis

    b = batch_ref[0]
    ids = lax.broadcasted_iota(jnp.int32, (N_GRAPHS, BN), 0)
    oh = (ids == b).astype(jnp.float32)
    psum = jnp.dot(oh, z, preferred_element_type=jnp.float32)
    pcnt = jnp.broadcast_to(jnp.sum(oh, axis=1, keepdims=True),
                            (N_GRAPHS, D_LAT))

    @pl.when(i == 0)
    def _():
        sums_ref[...] = jnp.zeros_like(sums_ref)
        cnt_ref[...] = jnp.zeros_like(cnt_ref)

    sums_ref[...] += psum
    cnt_ref[...] += pcnt

    @pl.when(i == GRID - 1)
    def _():
        agg_ref[...] = sums_ref[...] / jnp.maximum(cnt_ref[...], 1.0)


def _tc3(acc2, g2, dis8, b_e2, batch3):
    return pl.pallas_call(
        _tc3_body,
        grid=(GRID,),
        in_specs=[_acc_spec(D_LAT), _row_spec(D_LAT), _row_spec(_DDEG),
                  _full_spec((1, D_LAT)),
                  pl.BlockSpec((1, 1, BN), lambda i: (i, 0, 0))],
        out_specs=[_row_spec(D_LAT), _row_spec(D_LAT),
                   pl.BlockSpec((N_GRAPHS, D_LAT), lambda i: (0, 0))],
        out_shape=[jax.ShapeDtypeStruct((N, D_LAT), jnp.float32),
                   jax.ShapeDtypeStruct((N, D_LAT), jnp.float32),
                   jax.ShapeDtypeStruct((N_GRAPHS, D_LAT), jnp.float32)],
        scratch_shapes=[pltpu.VMEM((N_GRAPHS, D_LAT), jnp.float32),
                        pltpu.VMEM((N_GRAPHS, D_LAT), jnp.float32)],
    )(acc2, g2, dis8, b_e2.reshape(1, D_LAT), batch3)


def _tc4_body(acc_ref, g_ref, dis_ref, w_ref, b_ref, out_ref):
    dis = dis_ref[:, 0:1]
    t3 = dis * (acc_ref[0] + acc_ref[1] + g_ref[...])
    h2 = jnp.maximum(jnp.dot(t3, w_ref[...], preferred_element_type=jnp.float32)
                     + b_ref[0], 0.0)
    out_ref[...] = h2 * dis


def _tc4(acc3, g3, dis8, W_d1, b_d1):
    return pl.pallas_call(
        _tc4_body,
        grid=(GRID,),
        in_specs=[_acc_spec(D_LAT), _row_spec(D_LAT), _row_spec(_DDEG),
                  _full_spec((D_LAT, D_HID)), _full_spec((1, D_HID))],
        out_specs=[_row_spec(D_HID)],
        out_shape=[jax.ShapeDtypeStruct((N, D_HID), jnp.float32)],
    )(acc3, g3, dis8, W_d1, b_d1.reshape(1, D_HID))[0]


def _tc5_body(acc_ref, g_ref, dis_ref, w_ref, b_ref, out_ref):
    dis = dis_ref[:, 0:1]
    t4 = dis * (acc_ref[0] + acc_ref[1] + g_ref[...])
    out_ref[...] = (jnp.dot(t4, w_ref[...], preferred_element_type=jnp.float32)
                    + b_ref[0])


def _tc5(acc4, g4, dis8, W_d2, b_d2):
    return pl.pallas_call(
        _tc5_body,
        grid=(GRID,),
        in_specs=[_acc_spec(D_HID), _row_spec(D_HID), _row_spec(_DDEG),
                  _full_spec((D_HID, D_IN)), _full_spec((1, D_IN))],
        out_specs=[_row_spec(D_IN)],
        out_shape=[jax.ShapeDtypeStruct((N, D_IN), jnp.float32)],
    )(acc4, g4, dis8, W_d2, b_d2.reshape(1, D_IN))[0]


def kernel(x, edge_index, batch, W_e1, b_e1, W_e2, b_e2, W_d1, b_d1, W_d2, b_d2):
    src = edge_index[0]
    dst = edge_index[1]
    pad = PADE - E
    src3 = jnp.concatenate([src, jnp.zeros((pad,), src.dtype)]).reshape(NW, NCH, CH)
    dst3 = jnp.concatenate([dst, jnp.full((pad,), N, dst.dtype)]).reshape(NW, NCH, CH)
    ones8 = jnp.ones((CH, _DDEG), jnp.float32)
    batch3 = batch.reshape(GRID, 1, BN)

    scat64 = _make_sc_scatter(D_HID)
    scat32 = _make_sc_scatter(D_LAT)

    deg_part = _make_sc_degree()(dst3, ones8)
    g1, dis8 = _tc1(x, W_e1, deg_part)
    acc1 = scat64(src3, dst3, g1)
    g2 = _tc2(acc1, g1, dis8, b_e1, W_e2)
    acc2 = scat32(src3, dst3, g2)
    z, g3, agg = _tc3(acc2, g2, dis8, b_e2, batch3)
    acc3 = scat32(src3, dst3, g3)
    g4 = _tc4(acc3, g3, dis8, W_d1, b_d1)
    acc4 = scat64(src3, dst3, g4)
    x_hat = _tc5(acc4, g4, dis8, W_d2, b_d2)
    return (x_hat, z, agg)


# speedup vs baseline: 1.7943x; 1.7943x over previous
import functools

import jax
import jax.numpy as jnp
from jax import lax
from jax.experimental import pallas as pl
from jax.experimental.pallas import tpu as pltpu
from jax.experimental.pallas import tpu_sc as plsc

N = 10000
E = 320000
D_IN = 128
D_HID = 64
D_LAT = 32
N_GRAPHS = 16

NC = 2
NS = 16
NW = NC * NS
CH = 128
NCH = 80
K4 = 4
PADE = NW * NCH * CH
NACC = 10240
ZROWS = 64
BN = 1000
GRID = N // BN

@functools.lru_cache(maxsize=None)
def _get_mesh():
    return plsc.VectorSubcoreMesh(core_axis_name="c", subcore_axis_name="s",
                                  num_cores=NC, num_subcores=NS)


DHALF = 32


@functools.lru_cache(maxsize=None)
def _make_sc_scatter(H):
    rows_per_tile_out = NACC // NS
    zcopies = NACC // NS // ZROWS
    D = DHALF

    @functools.partial(
        pl.kernel,
        out_type=jax.ShapeDtypeStruct((NC, H, NACC, D), jnp.float32),
        mesh=_get_mesh(),
        compiler_params=pltpu.CompilerParams(use_tc_tiling_on_sc=False),
        scratch_types=[
            pltpu.VMEM((NCH, CH), jnp.int32),
            pltpu.VMEM((NCH, CH), jnp.int32),
            pltpu.VMEM((2 * K4 * CH, D), jnp.float32),
            pltpu.VMEM((ZROWS, D), jnp.float32),
            pltpu.VMEM_SHARED((NACC, D), jnp.float32),
            pltpu.VMEM_SHARED((N, D), jnp.float32),
            pltpu.SemaphoreType.DMA((2 * K4,)),
        ],
    )
    def k(src3, dst3, g2, out, src_v, dst_v, rows_v, zero_v, acc_sh, g_sh, gsem):
        c = lax.axis_index("c")
        s = lax.axis_index("s")
        wid = s * NC + c

        pltpu.sync_copy(src3.at[wid], src_v)
        pltpu.sync_copy(dst3.at[wid], dst_v)

        def zb(i, _):
            zero_v[i, :] = jnp.zeros((D,), jnp.float32)
            return 0
        lax.fori_loop(0, ZROWS, zb, 0)

        def buf(b):
            return rows_v.at[pl.ds(b * CH, CH)]

        def wait_g(b):
            pltpu.make_async_copy(g2.at[0, pl.ds(0, CH)], buf(b), gsem.at[b]).wait()

        def scat(j, b):
            pltpu.sync_copy(buf(b), acc_sh.at[dst_v.at[j]], add=True)

        g0 = s * (N // NS)
        r0 = s * rows_per_tile_out

        for h in range(H):
            pltpu.sync_copy(g2.at[h, pl.ds(g0, N // NS)],
                            g_sh.at[pl.ds(g0, N // NS)])

            def zc(t, _):
                pltpu.sync_copy(zero_v,
                                acc_sh.at[pl.ds(s * (NACC // NS) + t * ZROWS, ZROWS)])
                return 0
            lax.fori_loop(0, zcopies, zc, 0)
            plsc.subcore_barrier()

            def fire(j, b):
                pltpu.async_copy(g_sh.at[src_v.at[j]], buf(b), gsem.at[b])

            for b in range(K4):
                fire(b, b)

            @pl.loop(0, NCH, step=2 * K4)
            def _pair(j0):
                for b in range(K4):
                    fire(j0 + K4 + b, K4 + b)
                for b in range(K4):
                    wait_g(b)
                    scat(j0 + b, b)

                @pl.when(j0 + 2 * K4 < NCH)
                def _():
                    for b in range(K4):
                        fire(j0 + 2 * K4 + b, b)

                for b in range(K4):
                    wait_g(K4 + b)
                    scat(j0 + K4 + b, K4 + b)

            plsc.subcore_barrier()

            pltpu.sync_copy(acc_sh.at[pl.ds(r0, rows_per_tile_out)],
                            out.at[c, h, pl.ds(r0, rows_per_tile_out)])

    return k


_DDEG = 8


@functools.lru_cache(maxsize=None)
def _make_sc_degree():
    @functools.partial(
        pl.kernel,
        out_type=jax.ShapeDtypeStruct((NC, NACC, _DDEG), jnp.float32),
        mesh=_get_mesh(),
        compiler_params=pltpu.CompilerParams(use_tc_tiling_on_sc=False),
        scratch_types=[
            pltpu.VMEM((NCH, CH), jnp.int32),
            pltpu.VMEM((CH, _DDEG), jnp.float32),
            pltpu.VMEM((ZROWS, _DDEG), jnp.float32),
            pltpu.VMEM_SHARED((NACC, _DDEG), jnp.float32),
        ],
    )
    def k(dst3, ones_hbm, out, dst_v, ones_v, zero_v, acc_sh):
        c = lax.axis_index("c")
        s = lax.axis_index("s")
        wid = s * NC + c
        rows_per_tile_out = NACC // NS
        zcopies = NACC // NS // ZROWS

        pltpu.sync_copy(dst3.at[wid], dst_v)
        pltpu.sync_copy(ones_hbm, ones_v)

        def zb(i, _):
            zero_v[i, :] = jnp.zeros((_DDEG,), jnp.float32)
            return 0
        lax.fori_loop(0, ZROWS, zb, 0)

        def zc(t, _):
            pltpu.sync_copy(zero_v, acc_sh.at[pl.ds(s * (NACC // NS) + t * ZROWS, ZROWS)])
            return 0
        lax.fori_loop(0, zcopies, zc, 0)
        plsc.subcore_barrier()

        def body(j, _):
            pltpu.sync_copy(ones_v, acc_sh.at[dst_v.at[j]], add=True)
            return 0
        lax.fori_loop(0, NCH, body, 0)
        plsc.subcore_barrier()

        r0 = s * rows_per_tile_out
        pltpu.sync_copy(acc_sh.at[pl.ds(r0, rows_per_tile_out)],
                        out.at[c, pl.ds(r0, rows_per_tile_out)])

    return k


def _row_spec(d):
    return pl.BlockSpec((BN, d), lambda i: (i, 0))


def _split_spec(h):
    return pl.BlockSpec((h, BN, DHALF), lambda i: (0, i, 0))


def _acc_spec(h):
    return pl.BlockSpec((NC, h, BN, DHALF), lambda i: (0, 0, i, 0))


def _full_spec(shape):
    return pl.BlockSpec(shape, lambda i: (0,) * len(shape))


def _accsum(acc_ref, g_ref, h):
    return acc_ref[0, h] + acc_ref[1, h] + g_ref[h]


def _tc1_body(x_ref, w_ref, deg_ref, g1_ref, dis_ref):
    deg = deg_ref[0, :, 0:1] + deg_ref[1, :, 0:1] + 1.0
    dis = lax.rsqrt(deg)
    dis_ref[...] = jnp.broadcast_to(dis, (BN, _DDEG))
    h = jnp.dot(x_ref[...], w_ref[...], preferred_element_type=jnp.float32) * dis
    g1_ref[0] = h[:, :DHALF]
    g1_ref[1] = h[:, DHALF:]


def _tc1(x, W_e1, deg_part):
    return pl.pallas_call(
        _tc1_body,
        grid=(GRID,),
        in_specs=[_row_spec(D_IN), _full_spec((D_IN, D_HID)),
                  pl.BlockSpec((NC, BN, _DDEG), lambda i: (0, i, 0))],
        out_specs=[_split_spec(2), _row_spec(_DDEG)],
        out_shape=[jax.ShapeDtypeStruct((2, N, DHALF), jnp.float32),
                   jax.ShapeDtypeStruct((N, _DDEG), jnp.float32)],
    )(x, W_e1, deg_part)


def _tc2_body(acc_ref, g_ref, dis_ref, b_ref, w_ref, out_ref):
    dis = dis_ref[:, 0:1]
    h0 = jnp.maximum(dis * _accsum(acc_ref, g_ref, 0) + b_ref[0, :DHALF], 0.0)
    h1 = jnp.maximum(dis * _accsum(acc_ref, g_ref, 1) + b_ref[0, DHALF:], 0.0)
    out_ref[0] = (jnp.dot(h0, w_ref[:DHALF], preferred_element_type=jnp.float32)
                  + jnp.dot(h1, w_ref[DHALF:], preferred_element_type=jnp.float32)
                  ) * dis


def _tc2(acc1, g1, dis8, b_e1, W_e2):
    return pl.pallas_call(
        _tc2_body,
        grid=(GRID,),
        in_specs=[_acc_spec(2), _split_spec(2), _row_spec(_DDEG),
                  _full_spec((1, D_HID)), _full_spec((D_HID, D_LAT))],
        out_specs=[_split_spec(1)],
        out_shape=[jax.ShapeDtypeStruct((1, N, D_LAT), jnp.float32)],
    )(acc1, g1, dis8, b_e1.reshape(1, D_HID), W_e2)[0]


def _tc3_body(acc_ref, g_ref, dis_ref, b_ref, batch_ref,
              z_ref, g3_ref, agg_ref, sums_ref, cnt_ref):
    i = pl.program_id(0)
    dis = dis_ref[:, 0:1]
    z = dis * _accsum(acc_ref, g_ref, 0) + b_ref[0]
    z_ref[...] = z
    g3_ref[0] = z * dis

    b = batch_ref[0]
    ids = lax.broadcasted_iota(jnp.int32, (N_GRAPHS, BN), 0)
    oh = (ids == b).astype(jnp.float32)
    psum = jnp.dot(oh, z, preferred_element_type=jnp.float32)
    pcnt = jnp.broadcast_to(jnp.sum(oh, axis=1, keepdims=True),
                            (N_GRAPHS, D_LAT))

    @pl.when(i == 0)
    def _():
        sums_ref[...] = jnp.zeros_like(sums_ref)
        cnt_ref[...] = jnp.zeros_like(cnt_ref)

    sums_ref[...] += psum
    cnt_ref[...] += pcnt

    @pl.when(i == GRID - 1)
    def _():
        agg_ref[...] = sums_ref[...] / jnp.maximum(cnt_ref[...], 1.0)


def _tc3(acc2, g2, dis8, b_e2, batch3):
    return pl.pallas_call(
        _tc3_body,
        grid=(GRID,),
        in_specs=[_acc_spec(1), _split_spec(1), _row_spec(_DDEG),
                  _full_spec((1, D_LAT)),
                  pl.BlockSpec((1, 1, BN), lambda i: (i, 0, 0))],
        out_specs=[_row_spec(D_LAT), _split_spec(1),
                   pl.BlockSpec((N_GRAPHS, D_LAT), lambda i: (0, 0))],
        out_shape=[jax.ShapeDtypeStruct((N, D_LAT), jnp.float32),
                   jax.ShapeDtypeStruct((1, N, D_LAT), jnp.float32),
                   jax.ShapeDtypeStruct((N_GRAPHS, D_LAT), jnp.float32)],
        scratch_shapes=[pltpu.VMEM((N_GRAPHS, D_LAT), jnp.float32),
                        pltpu.VMEM((N_GRAPHS, D_LAT), jnp.float32)],
    )(acc2, g2, dis8, b_e2.reshape(1, D_LAT), batch3)


def _tc4_body(acc_ref, g_ref, dis_ref, w_ref, b_ref, out_ref):
    dis = dis_ref[:, 0:1]
    t3 = dis * _accsum(acc_ref, g_ref, 0)
    h2 = jnp.maximum(jnp.dot(t3, w_ref[...], preferred_element_type=jnp.float32)
                     + b_ref[0], 0.0)
    g4 = h2 * dis
    out_ref[0] = g4[:, :DHALF]
    out_ref[1] = g4[:, DHALF:]


def _tc4(acc3, g3, dis8, W_d1, b_d1):
    return pl.pallas_call(
        _tc4_body,
        grid=(GRID,),
        in_specs=[_acc_spec(1), _split_spec(1), _row_spec(_DDEG),
                  _full_spec((D_LAT, D_HID)), _full_spec((1, D_HID))],
        out_specs=[_split_spec(2)],
        out_shape=[jax.ShapeDtypeStruct((2, N, DHALF), jnp.float32)],
    )(acc3, g3, dis8, W_d1, b_d1.reshape(1, D_HID))[0]


def _tc5_body(acc_ref, g_ref, dis_ref, w_ref, b_ref, out_ref):
    dis = dis_ref[:, 0:1]
    t0 = dis * _accsum(acc_ref, g_ref, 0)
    t1 = dis * _accsum(acc_ref, g_ref, 1)
    out_ref[...] = (jnp.dot(t0, w_ref[:DHALF], preferred_element_type=jnp.float32)
                    + jnp.dot(t1, w_ref[DHALF:], preferred_element_type=jnp.float32)
                    + b_ref[0])


def _tc5(acc4, g4, dis8, W_d2, b_d2):
    return pl.pallas_call(
        _tc5_body,
        grid=(GRID,),
        in_specs=[_acc_spec(2), _split_spec(2), _row_spec(_DDEG),
                  _full_spec((D_HID, D_IN)), _full_spec((1, D_IN))],
        out_specs=[_row_spec(D_IN)],
        out_shape=[jax.ShapeDtypeStruct((N, D_IN), jnp.float32)],
    )(acc4, g4, dis8, W_d2, b_d2.reshape(1, D_IN))[0]


def kernel(x, edge_index, batch, W_e1, b_e1, W_e2, b_e2, W_d1, b_d1, W_d2, b_d2):
    src = edge_index[0]
    dst = edge_index[1]
    pad = PADE - E
    src3 = jnp.concatenate([src, jnp.zeros((pad,), src.dtype)]).reshape(NW, NCH, CH)
    dst3 = jnp.concatenate([dst, jnp.full((pad,), N, dst.dtype)]).reshape(NW, NCH, CH)
    ones8 = jnp.ones((CH, _DDEG), jnp.float32)
    batch3 = batch.reshape(GRID, 1, BN)

    scat2 = _make_sc_scatter(2)
    scat1 = _make_sc_scatter(1)

    deg_part = _make_sc_degree()(dst3, ones8)
    g1, dis8 = _tc1(x, W_e1, deg_part)
    acc1 = scat2(src3, dst3, g1)
    g2 = _tc2(acc1, g1, dis8, b_e1, W_e2)
    acc2 = scat1(src3, dst3, g2)
    z, g3, agg = _tc3(acc2, g2, dis8, b_e2, batch3)
    acc3 = scat1(src3, dst3, g3)
    g4 = _tc4(acc3, g3, dis8, W_d1, b_d1)
    acc4 = scat2(src3, dst3, g4)
    x_hat = _tc5(acc4, g4, dis8, W_d2, b_d2)
    return (x_hat, z, agg)
